# baseline (speedup 1.0000x reference)
import jax
import jax.numpy as jnp
from jax import lax
from jax.experimental import pallas as pl
from jax.experimental.pallas import tpu as pltpu
from jax.experimental.pallas import tpu_sc as plsc

VOCAB = 1_000_000
D = 64
SEQ = 200
BATCH = 1024
ROWS = BATCH * SEQ

NC = 2
NS = 16
LANES = 16
NW = NC * NS

SEQ_PER_W = BATCH // NW
SPLITS = ((0, 128), (128, 72))

SCALE = 8.0


def _body(x_ref, tab_ref, pos_ref, out_ref, idx_v, pos_v, rows_v, sem):
    wid = lax.axis_index("s") * NC + lax.axis_index("c")
    base_seq = wid * SEQ_PER_W

    pltpu.sync_copy(x_ref.at[pl.ds(base_seq * SEQ, SEQ_PER_W * SEQ)], idx_v)
    pltpu.sync_copy(pos_ref, pos_v)

    def per_seq(s, carry):
        row0 = s * SEQ
        for off, n in SPLITS:
            pltpu.async_copy(
                tab_ref.at[idx_v.at[pl.ds(row0 + off, n)]],
                rows_v.at[pl.ds(off, n)],
                sem,
            ).wait()

        def per_row(r, c2):
            for c in range(D // LANES):
                sl = pl.ds(c * LANES, LANES)
                rows_v[r, sl] = rows_v[r, sl] * SCALE + pos_v[r, sl]
            return c2

        lax.fori_loop(0, SEQ, per_row, 0, unroll=2)

        pltpu.sync_copy(
            rows_v, out_ref.at[pl.ds((base_seq + s) * SEQ, SEQ)]
        )
        return carry

    lax.fori_loop(0, SEQ_PER_W, per_seq, 0)


@jax.jit
def _embed(x_flat, token_table, pos_table):
    mesh = plsc.VectorSubcoreMesh(
        core_axis_name="c", subcore_axis_name="s", num_cores=NC,
        num_subcores=NS,
    )
    f = pl.kernel(
        _body,
        out_type=jax.ShapeDtypeStruct((ROWS, D), jnp.float32),
        mesh=mesh,
        scratch_types=[
            pltpu.VMEM((SEQ_PER_W * SEQ,), jnp.int32),
            pltpu.VMEM((SEQ, D), jnp.float32),
            pltpu.VMEM((SEQ, D), jnp.float32),
            pltpu.SemaphoreType.DMA,
        ],
        compiler_params=pltpu.CompilerParams(use_tc_tiling_on_sc=False),
    )
    return f(x_flat, token_table, pos_table)


def kernel(x, token_table, pos_table):
    x_flat = x.reshape(ROWS).astype(jnp.int32)
    out = _embed(x_flat, token_table, pos_table)
    return out.reshape(BATCH, SEQ, D)


# speedup vs baseline: 1.1815x; 1.1815x over previous
import jax
import jax.numpy as jnp
from jax import lax
from jax.experimental import pallas as pl
from jax.experimental.pallas import tpu as pltpu
from jax.experimental.pallas import tpu_sc as plsc

VOCAB = 1_000_000
D = 64
SEQ = 200
BATCH = 1024
ROWS = BATCH * SEQ

NC = 2
NS = 16
LANES = 16
NW = NC * NS

SEQ_PER_W = BATCH // NW
SPLITS = ((0, 128), (128, 72))

SCALE = 8.0


def _body(x_ref, tab_ref, pos_ref, out_ref, idx_v, pos_v, rows0, rows1,
          gsems, ssems):
    wid = lax.axis_index("s") * NC + lax.axis_index("c")
    base_seq = wid * SEQ_PER_W
    bufs = (rows0, rows1)

    pltpu.sync_copy(x_ref.at[pl.ds(base_seq * SEQ, SEQ_PER_W * SEQ)], idx_v)
    pltpu.sync_copy(pos_ref, pos_v)

    def issue_gather(s, b):
        row0 = s * SEQ
        for off, n in SPLITS:
            pltpu.async_copy(
                tab_ref.at[idx_v.at[pl.ds(row0 + off, n)]],
                bufs[b].at[pl.ds(off, n)],
                gsems.at[b],
            )

    def drain(sem, b):
        pltpu.make_async_copy(out_ref.at[pl.ds(0, SEQ)], bufs[b], sem).wait()

    def compute(b):
        buf = bufs[b]

        @plsc.parallel_loop(0, SEQ, unroll=4)
        def _(r):
            for c in range(D // LANES):
                sl = pl.ds(c * LANES, LANES)
                buf[r, sl] = buf[r, sl] * SCALE + pos_v[r, sl]

    issue_gather(0, 0)

    @pl.loop(0, SEQ_PER_W, step=2)
    def _(g):
        for b in range(2):
            s = g + b
            nb = 1 - b

            @pl.when(s + 1 < SEQ_PER_W)
            def _():
                @pl.when(s >= 1)
                def _():
                    drain(ssems.at[nb], nb)

                issue_gather(s + 1, nb)

            drain(gsems.at[b], b)
            compute(b)
            pltpu.async_copy(
                bufs[b], out_ref.at[pl.ds((base_seq + s) * SEQ, SEQ)],
                ssems.at[b],
            )

    drain(ssems.at[0], 0)
    drain(ssems.at[1], 1)


@jax.jit
def _embed(x_flat, token_table, pos_table):
    mesh = plsc.VectorSubcoreMesh(
        core_axis_name="c", subcore_axis_name="s", num_cores=NC,
        num_subcores=NS,
    )
    f = pl.kernel(
        _body,
        out_type=jax.ShapeDtypeStruct((ROWS, D), jnp.float32),
        mesh=mesh,
        scratch_types=[
            pltpu.VMEM((SEQ_PER_W * SEQ,), jnp.int32),
            pltpu.VMEM((SEQ, D), jnp.float32),
            pltpu.VMEM((SEQ, D), jnp.float32),
            pltpu.VMEM((SEQ, D), jnp.float32),
            pltpu.SemaphoreType.DMA((2,)),
            pltpu.SemaphoreType.DMA((2,)),
        ],
        compiler_params=pltpu.CompilerParams(use_tc_tiling_on_sc=False),
    )
    return f(x_flat, token_table, pos_table)


def kernel(x, token_table, pos_table):
    x_flat = x.reshape(ROWS).astype(jnp.int32)
    out = _embed(x_flat, token_table, pos_table)
    return out.reshape(BATCH, SEQ, D)
